# Initial kernel scaffold; baseline (speedup 1.0000x reference)
import jax
import jax.numpy as jnp
from jax.experimental import pallas as pl


def kernel(y, feat, centers):
    raise NotImplementedError("write your pallas kernel here")


# speedup vs baseline: 5.5809x; 5.5809x over previous
import functools

import jax
import jax.numpy as jnp
from jax import lax
from jax.experimental import pallas as pl
from jax.experimental.pallas import tpu as pltpu

_NUM_CLASSES = 1000
_FEAT = 128
_BATCH = 16384
_CPAD = 1024
_BB = 512


def _body(y_ref, feat_ref, centers_ref, out_ref, accm_ref, accv_ref):
    i = pl.program_id(0)
    nsteps = pl.num_programs(0)

    @pl.when(i == 0)
    def _init():
        accm_ref[...] = jnp.zeros_like(accm_ref)
        accv_ref[...] = jnp.zeros_like(accv_ref)

    yb = y_ref[0]
    fb = feat_ref[...]

    cls = lax.broadcasted_iota(jnp.int32, (_CPAD, _BB), 0)
    ohT = jnp.where(cls == jnp.broadcast_to(yb, (_CPAD, _BB)), 1.0, 0.0)

    accm_ref[...] += jnp.dot(ohT, fb, preferred_element_type=jnp.float32)

    q = jnp.sum(fb * fb, axis=1, keepdims=True)
    lane = lax.broadcasted_iota(jnp.int32, (_BB, _FEAT), 1)
    u = jnp.where(lane == 0, q, jnp.where(lane == 1, 1.0, 0.0))
    accv_ref[...] += jnp.dot(ohT, u, preferred_element_type=jnp.float32)

    @pl.when(i == nsteps - 1)
    def _fini():
        C = centers_ref[...]
        m = accm_ref[...]
        S2 = accv_ref[:, 0]
        n = accv_ref[:, 1]
        z = jnp.sum(C * C, axis=1)
        d = jnp.sum(m * C, axis=1)
        num = S2 - 2.0 * d + n * z
        out_ref[...] = jnp.reshape(0.5 * jnp.sum(num / (n + 1.0)), (1, 1))


def kernel(y, feat, centers):
    y3 = y.astype(jnp.int32).reshape(_BATCH // _BB, 1, _BB)
    cpad = jnp.pad(centers, ((0, _CPAD - _NUM_CLASSES), (0, 0)))
    grid = (_BATCH // _BB,)
    out = pl.pallas_call(
        _body,
        grid=grid,
        in_specs=[
            pl.BlockSpec((1, 1, _BB), lambda i: (i, 0, 0)),
            pl.BlockSpec((_BB, _FEAT), lambda i: (i, 0)),
            pl.BlockSpec((_CPAD, _FEAT), lambda i: (0, 0)),
        ],
        out_specs=pl.BlockSpec((1, 1), lambda i: (0, 0)),
        out_shape=jax.ShapeDtypeStruct((1, 1), jnp.float32),
        scratch_shapes=[
            pltpu.VMEM((_CPAD, _FEAT), jnp.float32),
            pltpu.VMEM((_CPAD, _FEAT), jnp.float32),
        ],
    )(y3, feat, cpad)
    return out[0, 0]
